# baseline (speedup 1.0000x reference)
import jax
import jax.numpy as jnp
from jax import lax
from jax.experimental import pallas as pl
from jax.experimental.pallas import tpu as pltpu
from jax.experimental.pallas import tpu_sc as plsc

_BATCH = 16384
_FEAT = 1000000
_FIELD = 26
_K = 16
_NC = 2
_NS = 16
_NW = _NC * _NS
_ROWS_W = _BATCH // _NW
_CH = 128
_NCH = _ROWS_W // _CH
_ICH = _CH * _FIELD

_TBLK = 8192
_W = _TBLK // 8
_NBLK = -(-_FEAT // _TBLK)
_FEATP = _NBLK * _TBLK


def _pk_body(src_ref, dst_ref):
    eye = jnp.eye(_K, dtype=jnp.float32)
    parts = []
    for u in range(8):
        xs = src_ref[:, u * _W:(u + 1) * _W]
        parts.append(jax.lax.dot_general(
            xs, eye, (((0,), (0,)), ((), ())),
            preferred_element_type=jnp.float32))
    dst_ref[...] = jnp.concatenate(parts, axis=1)


def _repack_fm(fm_t):
    out = pl.pallas_call(
        _pk_body,
        grid=(_NBLK,),
        in_specs=[pl.BlockSpec((_K, _TBLK), lambda i: (0, i))],
        out_specs=pl.BlockSpec((_W, 128), lambda i: (i, 0)),
        out_shape=jax.ShapeDtypeStruct((_FEATP * _K // 128, 128),
                                       jnp.float32),
    )(fm_t)
    return out.reshape(_FEATP, _K)


def _fm_body(idx_hbm, fv_hbm, lw_hbm, bias_hbm, fm_hbm, out_hbm,
             idx_v, idx2_v, fv_v, rows_v, lin_v, z_v, bias_v, gsem, lsem):
    wid = lax.axis_index("s") * _NC + lax.axis_index("c")
    base = wid * _ROWS_W

    pltpu.sync_copy(bias_hbm, bias_v)
    bias_vec = bias_v[:]
    iota = lax.iota(jnp.int32, _K)
    ksplat = [jnp.full((_K,), k, jnp.int32) for k in range(_K)]

    def chunk_body(c, carry):
        rbase = base + c * _CH

        pltpu.sync_copy(idx_hbm.at[:, pl.ds(rbase, _CH)], idx_v)
        pltpu.sync_copy(fv_hbm.at[:, pl.ds(rbase, _CH)], fv_v)

        def xform(f, carry2):
            for g in range(_CH // _K):
                t = idx_v[f, pl.ds(g * _K, _K)]
                rp = ((t & ~(_TBLK - 1))
                      | ((t & (_W - 1)) << 3)
                      | ((t >> 10) & 7))
                idx2_v[f, pl.ds(g * _K, _K)] = rp
            return carry2

        lax.fori_loop(0, _FIELD, xform, 0)

        def fire(f, carry2):
            pltpu.async_copy(lw_hbm.at[idx_v.at[f]],
                             lin_v.at[pl.ds(f * _CH, _CH)], lsem)
            pltpu.async_copy(fm_hbm.at[idx2_v.at[f]],
                             rows_v.at[pl.ds(f * _CH, _CH)], gsem)
            return carry2

        lax.fori_loop(0, _FIELD, fire, 0)
        pltpu.make_async_copy(fm_hbm.at[pl.ds(0, _ICH)], rows_v,
                              gsem).wait()
        pltpu.make_async_copy(lw_hbm.at[pl.ds(0, _ICH)], lin_v,
                              lsem).wait()

        def blk_body(blk, carry2):
            r0 = blk * _K
            rids = r0 + iota
            zero = jnp.zeros((_K,), jnp.float32)
            s = [zero] * _K
            q = [zero] * 4
            lin = zero
            for f in range(_FIELD):
                off = f * _CH
                fvv = fv_v[f, pl.ds(r0, _K)]
                lv = lin_v[pl.ds(off + r0, _K)]
                lin = lin + lv * fvv
                ridx = rids + off
                for k in range(_K):
                    xg = plsc.load_gather(rows_v, [ridx, ksplat[k]])
                    xv = xg * fvv
                    s[k] = s[k] + xv
                    q[k % 4] = q[k % 4] + xv * xv
            p = [s[k] * s[k] for k in range(_K)]
            while len(p) > 1:
                p = [p[i] + p[i + 1] for i in range(0, len(p), 2)]
            z = 0.5 * (p[0] - (q[0] + q[1] + q[2] + q[3])) + lin + bias_vec
            z = 1.0 / (1.0 + jnp.exp(-z))
            z_v[pl.ds(r0, _K)] = z
            return carry2

        lax.fori_loop(0, _CH // _K, blk_body, 0)
        pltpu.sync_copy(z_v, out_hbm.at[pl.ds(rbase, _CH)])
        return carry

    lax.fori_loop(0, _NCH, chunk_body, 0)


@jax.jit
def _fm_sc(idx_t, fv_t, lw_flat, bias16, fm_pk):
    mesh = plsc.VectorSubcoreMesh(core_axis_name="c", subcore_axis_name="s",
                                  num_cores=_NC, num_subcores=_NS)
    f = pl.kernel(
        _fm_body,
        out_type=jax.ShapeDtypeStruct((_BATCH,), jnp.float32),
        mesh=mesh,
        compiler_params=pltpu.CompilerParams(needs_layout_passes=False,
                                             use_tc_tiling_on_sc=False),
        scratch_types=[
            pltpu.VMEM((_FIELD, _CH), jnp.int32),
            pltpu.VMEM((_FIELD, _CH), jnp.int32),
            pltpu.VMEM((_FIELD, _CH), jnp.float32),
            pltpu.VMEM((_ICH, _K), jnp.float32),
            pltpu.VMEM((_ICH,), jnp.float32),
            pltpu.VMEM((_CH,), jnp.float32),
            pltpu.VMEM((_K,), jnp.float32),
            pltpu.SemaphoreType.DMA,
            pltpu.SemaphoreType.DMA,
        ],
    )
    return f(idx_t, fv_t, lw_flat, bias16, fm_pk)


@jax.jit
def _fm_full(feat_index, feat_value, linear_weight, linear_bias, fm_weight):
    idx_t = feat_index.T
    fv_t = feat_value.T
    lw_flat = linear_weight.T.reshape(_FEAT)
    bias16 = jnp.broadcast_to(linear_bias.reshape(()), (_K,))
    fm_pk = _repack_fm(fm_weight.T)
    out = _fm_sc(idx_t, fv_t, lw_flat, bias16, fm_pk)
    return out.reshape(_BATCH, 1)


def kernel(feat_index, feat_value, linear_weight, linear_bias, fm_weight):
    return _fm_full(feat_index, feat_value, linear_weight, linear_bias,
                    fm_weight)


# speedup vs baseline: 1.3462x; 1.3462x over previous
import jax
import jax.numpy as jnp
from jax import lax
from jax.experimental import pallas as pl
from jax.experimental.pallas import tpu as pltpu
from jax.experimental.pallas import tpu_sc as plsc

_BATCH = 16384
_FEAT = 1000000
_FIELD = 26
_K = 16
_NC = 2
_NS = 16
_NW = _NC * _NS
_ROWS_W = _BATCH // _NW
_CH = 128
_NCH = _ROWS_W // _CH
_ICH = _CH * _FIELD

_TBLK = 8192
_W = _TBLK // 8
_NBLK = -(-_FEAT // _TBLK)
_FEATP = _NBLK * _TBLK


def _pk_body(src_ref, dst_ref):
    rows = lax.broadcasted_iota(jnp.int32, (_K, 128), 0)
    lanes = lax.broadcasted_iota(jnp.int32, (_K, 128), 1)
    acc = None
    for u in range(8):
        xs = src_ref[:, u * _W:(u + 1) * _W]
        sel = (lanes == rows + _K * u).astype(jnp.float32)
        t = jax.lax.dot_general(xs, sel, (((0,), (0,)), ((), ())),
                                preferred_element_type=jnp.float32)
        acc = t if acc is None else acc + t
    dst_ref[...] = acc


def _repack_fm(fm_t):
    out = pl.pallas_call(
        _pk_body,
        grid=(_NBLK,),
        in_specs=[pl.BlockSpec((_K, _TBLK), lambda i: (0, i))],
        out_specs=pl.BlockSpec((_W, 128), lambda i: (i, 0)),
        out_shape=jax.ShapeDtypeStruct((_FEATP * _K // 128, 128),
                                       jnp.float32),
    )(fm_t)
    return out.reshape(_FEATP, _K)


def _fm_body(idx_hbm, fv_hbm, lw_hbm, bias_hbm, fm_hbm, out_hbm,
             idx_v, idx2_v, fv_v, rows_v, lin_v, z_v, bias_v, gsem, lsem):
    wid = lax.axis_index("s") * _NC + lax.axis_index("c")
    base = wid * _ROWS_W

    pltpu.sync_copy(bias_hbm, bias_v)
    bias_vec = bias_v[:]
    iota = lax.iota(jnp.int32, _K)
    ksplat = [jnp.full((_K,), k, jnp.int32) for k in range(_K)]

    def chunk_body(c, carry):
        rbase = base + c * _CH

        pltpu.sync_copy(idx_hbm.at[:, pl.ds(rbase, _CH)], idx_v)
        pltpu.sync_copy(fv_hbm.at[:, pl.ds(rbase, _CH)], fv_v)

        def xform(f, carry2):
            for g in range(_CH // _K):
                t = idx_v[f, pl.ds(g * _K, _K)]
                rp = ((t & ~(_TBLK - 1))
                      | ((t & (_W - 1)) << 3)
                      | ((t >> 10) & 7))
                idx2_v[f, pl.ds(g * _K, _K)] = rp
            return carry2

        lax.fori_loop(0, _FIELD, xform, 0)

        def fire(f, carry2):
            pltpu.async_copy(lw_hbm.at[idx_v.at[f]],
                             lin_v.at[pl.ds(f * _CH, _CH)], lsem)
            pltpu.async_copy(fm_hbm.at[idx2_v.at[f]],
                             rows_v.at[pl.ds(f * _CH, _CH)], gsem)
            return carry2

        lax.fori_loop(0, _FIELD, fire, 0)
        pltpu.make_async_copy(fm_hbm.at[pl.ds(0, _ICH)], rows_v,
                              gsem).wait()
        pltpu.make_async_copy(lw_hbm.at[pl.ds(0, _ICH)], lin_v,
                              lsem).wait()

        def blk_body(blk, carry2):
            r0 = blk * _K
            rids = r0 + iota
            zero = jnp.zeros((_K,), jnp.float32)
            s = [zero] * _K
            q = [zero] * 4
            lin = zero
            for f in range(_FIELD):
                off = f * _CH
                fvv = fv_v[f, pl.ds(r0, _K)]
                lv = lin_v[pl.ds(off + r0, _K)]
                lin = lin + lv * fvv
                ridx = rids + off
                for k in range(_K):
                    xg = plsc.load_gather(rows_v, [ridx, ksplat[k]])
                    xv = xg * fvv
                    s[k] = s[k] + xv
                    q[k % 4] = q[k % 4] + xv * xv
            p = [s[k] * s[k] for k in range(_K)]
            while len(p) > 1:
                p = [p[i] + p[i + 1] for i in range(0, len(p), 2)]
            z = 0.5 * (p[0] - (q[0] + q[1] + q[2] + q[3])) + lin + bias_vec
            z = 1.0 / (1.0 + jnp.exp(-z))
            z_v[pl.ds(r0, _K)] = z
            return carry2

        lax.fori_loop(0, _CH // _K, blk_body, 0)
        pltpu.sync_copy(z_v, out_hbm.at[pl.ds(rbase, _CH)])
        return carry

    lax.fori_loop(0, _NCH, chunk_body, 0)


@jax.jit
def _fm_sc(idx_t, fv_t, lw_flat, bias16, fm_pk):
    mesh = plsc.VectorSubcoreMesh(core_axis_name="c", subcore_axis_name="s",
                                  num_cores=_NC, num_subcores=_NS)
    f = pl.kernel(
        _fm_body,
        out_type=jax.ShapeDtypeStruct((_BATCH,), jnp.float32),
        mesh=mesh,
        compiler_params=pltpu.CompilerParams(needs_layout_passes=False,
                                             use_tc_tiling_on_sc=False),
        scratch_types=[
            pltpu.VMEM((_FIELD, _CH), jnp.int32),
            pltpu.VMEM((_FIELD, _CH), jnp.int32),
            pltpu.VMEM((_FIELD, _CH), jnp.float32),
            pltpu.VMEM((_ICH, _K), jnp.float32),
            pltpu.VMEM((_ICH,), jnp.float32),
            pltpu.VMEM((_CH,), jnp.float32),
            pltpu.VMEM((_K,), jnp.float32),
            pltpu.SemaphoreType.DMA,
            pltpu.SemaphoreType.DMA,
        ],
    )
    return f(idx_t, fv_t, lw_flat, bias16, fm_pk)


@jax.jit
def _fm_full(feat_index, feat_value, linear_weight, linear_bias, fm_weight):
    idx_t = feat_index.T
    fv_t = feat_value.T
    lw_flat = linear_weight.T.reshape(_FEAT)
    bias16 = jnp.broadcast_to(linear_bias.reshape(()), (_K,))
    fm_pk = _repack_fm(fm_weight.T)
    out = _fm_sc(idx_t, fv_t, lw_flat, bias16, fm_pk)
    return out.reshape(_BATCH, 1)


def kernel(feat_index, feat_value, linear_weight, linear_bias, fm_weight):
    return _fm_full(feat_index, feat_value, linear_weight, linear_bias,
                    fm_weight)


# speedup vs baseline: 1.6891x; 1.2548x over previous
import jax
import jax.numpy as jnp
from jax import lax
from jax.experimental import pallas as pl
from jax.experimental.pallas import tpu as pltpu
from jax.experimental.pallas import tpu_sc as plsc

_BATCH = 16384
_FEAT = 1000000
_FIELD = 26
_K = 16
_NC = 2
_NS = 16
_NW = _NC * _NS
_ROWS_W = _BATCH // _NW
_CH = 128
_NCH = _ROWS_W // _CH
_ICH = _CH * _FIELD

_TBLK = 8192
_W = _TBLK // 8
_NBLK = -(-_FEAT // _TBLK)
_FEATP = _NBLK * _TBLK


def _pk_body(src_ref, dst_ref):
    x8 = jnp.concatenate(
        [src_ref[:, u * _W:(u + 1) * _W] for u in range(8)], axis=0)
    eye = jnp.eye(128, dtype=jnp.float32)
    dst_ref[...] = jax.lax.dot_general(
        x8, eye, (((0,), (0,)), ((), ())),
        preferred_element_type=jnp.float32)


def _repack_fm(fm_t):
    out = pl.pallas_call(
        _pk_body,
        grid=(_NBLK,),
        in_specs=[pl.BlockSpec((_K, _TBLK), lambda i: (0, i))],
        out_specs=pl.BlockSpec((_W, 128), lambda i: (i, 0)),
        out_shape=jax.ShapeDtypeStruct((_FEATP * _K // 128, 128),
                                       jnp.float32),
    )(fm_t)
    return out.reshape(_FEATP, _K)


def _fm_body(idx_hbm, fv_hbm, lw_hbm, bias_hbm, fm_hbm, out_hbm,
             idx_v, idx2_v, fv_v, rows_v, lin_v, z_v, bias_v, gsem, lsem):
    wid = lax.axis_index("s") * _NC + lax.axis_index("c")
    base = wid * _ROWS_W

    pltpu.sync_copy(bias_hbm, bias_v)
    bias_vec = bias_v[:]
    iota = lax.iota(jnp.int32, _K)
    ksplat = [jnp.full((_K,), k, jnp.int32) for k in range(_K)]

    def chunk_body(c, carry):
        rbase = base + c * _CH

        pltpu.sync_copy(idx_hbm.at[:, pl.ds(rbase, _CH)], idx_v)
        pltpu.sync_copy(fv_hbm.at[:, pl.ds(rbase, _CH)], fv_v)

        def xform(f, carry2):
            for g in range(_CH // _K):
                t = idx_v[f, pl.ds(g * _K, _K)]
                rp = ((t & ~(_TBLK - 1))
                      | ((t & (_W - 1)) << 3)
                      | ((t >> 10) & 7))
                idx2_v[f, pl.ds(g * _K, _K)] = rp
            return carry2

        lax.fori_loop(0, _FIELD, xform, 0)

        def fire(f, carry2):
            pltpu.async_copy(lw_hbm.at[idx_v.at[f]],
                             lin_v.at[pl.ds(f * _CH, _CH)], lsem)
            pltpu.async_copy(fm_hbm.at[idx2_v.at[f]],
                             rows_v.at[pl.ds(f * _CH, _CH)], gsem)
            return carry2

        lax.fori_loop(0, _FIELD, fire, 0)
        pltpu.make_async_copy(fm_hbm.at[pl.ds(0, _ICH)], rows_v,
                              gsem).wait()
        pltpu.make_async_copy(lw_hbm.at[pl.ds(0, _ICH)], lin_v,
                              lsem).wait()

        def blk_body(blk, carry2):
            r0 = blk * _K
            rids = r0 + iota
            zero = jnp.zeros((_K,), jnp.float32)
            s = [zero] * _K
            q = [zero] * 4
            lin = zero
            for f in range(_FIELD):
                off = f * _CH
                fvv = fv_v[f, pl.ds(r0, _K)]
                lv = lin_v[pl.ds(off + r0, _K)]
                lin = lin + lv * fvv
                ridx = rids + off
                for k in range(_K):
                    xg = plsc.load_gather(rows_v, [ridx, ksplat[k]])
                    xv = xg * fvv
                    s[k] = s[k] + xv
                    q[k % 4] = q[k % 4] + xv * xv
            p = [s[k] * s[k] for k in range(_K)]
            while len(p) > 1:
                p = [p[i] + p[i + 1] for i in range(0, len(p), 2)]
            z = 0.5 * (p[0] - (q[0] + q[1] + q[2] + q[3])) + lin + bias_vec
            z = 1.0 / (1.0 + jnp.exp(-z))
            z_v[pl.ds(r0, _K)] = z
            return carry2

        lax.fori_loop(0, _CH // _K, blk_body, 0)
        pltpu.sync_copy(z_v, out_hbm.at[pl.ds(rbase, _CH)])
        return carry

    lax.fori_loop(0, _NCH, chunk_body, 0)


@jax.jit
def _fm_sc(idx_t, fv_t, lw_flat, bias16, fm_pk):
    mesh = plsc.VectorSubcoreMesh(core_axis_name="c", subcore_axis_name="s",
                                  num_cores=_NC, num_subcores=_NS)
    f = pl.kernel(
        _fm_body,
        out_type=jax.ShapeDtypeStruct((_BATCH,), jnp.float32),
        mesh=mesh,
        compiler_params=pltpu.CompilerParams(needs_layout_passes=False,
                                             use_tc_tiling_on_sc=False),
        scratch_types=[
            pltpu.VMEM((_FIELD, _CH), jnp.int32),
            pltpu.VMEM((_FIELD, _CH), jnp.int32),
            pltpu.VMEM((_FIELD, _CH), jnp.float32),
            pltpu.VMEM((_ICH, _K), jnp.float32),
            pltpu.VMEM((_ICH,), jnp.float32),
            pltpu.VMEM((_CH,), jnp.float32),
            pltpu.VMEM((_K,), jnp.float32),
            pltpu.SemaphoreType.DMA,
            pltpu.SemaphoreType.DMA,
        ],
    )
    return f(idx_t, fv_t, lw_flat, bias16, fm_pk)


@jax.jit
def _fm_full(feat_index, feat_value, linear_weight, linear_bias, fm_weight):
    idx_t = feat_index.T
    fv_t = feat_value.T
    lw_flat = linear_weight.T.reshape(_FEAT)
    bias16 = jnp.broadcast_to(linear_bias.reshape(()), (_K,))
    fm_pk = _repack_fm(fm_weight.T)
    out = _fm_sc(idx_t, fv_t, lw_flat, bias16, fm_pk)
    return out.reshape(_BATCH, 1)


def kernel(feat_index, feat_value, linear_weight, linear_bias, fm_weight):
    return _fm_full(feat_index, feat_value, linear_weight, linear_bias,
                    fm_weight)
